# Initial kernel scaffold; baseline (speedup 1.0000x reference)
import jax
import jax.numpy as jnp
from jax.experimental import pallas as pl


def kernel(x_idx, edge_index, edge_attr_idx, batch, node_emb, edge_emb, nW1, nG1, nB1, nW2, nG2, nB2, eW1, eG1, eB1, eW2, eG2, eB2, eps_arr, L1W1, L1g1, L1b1, L1W2, L1g2, L1b2, L2W1, L2g1, L2b1, L2W2, L2g2, L2b2, Wf):
    raise NotImplementedError("write your pallas kernel here")


# speedup vs baseline: 1.4788x; 1.4788x over previous
import functools

import jax
import jax.numpy as jnp
from jax import lax
from jax.experimental import pallas as pl
from jax.experimental.pallas import tpu as pltpu
from jax.experimental.pallas import tpu_sc as plsc

N = 10000
E = 160000
H = 128
NG = 256
NCORES = 2
NSUB = 16


def _mm3_body(x_ref, a0_ref, a1_ref, c_ref, w_ref, h_ref):
    t = x_ref[...] * c_ref[...] + a0_ref[...] + a1_ref[...]
    h_ref[...] = jnp.dot(t, w_ref[...], preferred_element_type=jnp.float32)


def _armm_body(x_ref, m_in_ref, v_ref, g_ref, b_ref, w_ref, h_ref):
    t = jnp.maximum(
        g_ref[...] * (x_ref[...] - m_in_ref[...])
        / jnp.sqrt(v_ref[...] + 1e-5) + b_ref[...], 0.0)
    h_ref[...] = jnp.dot(t, w_ref[...], preferred_element_type=jnp.float32)


def _mm3(x, a0, a1, c, w, br):
    r = x.shape[0]
    row = lambda i: (i, 0)
    fixed = lambda i: (0, 0)
    return pl.pallas_call(
        _mm3_body,
        grid=(r // br,),
        in_specs=[pl.BlockSpec((br, H), row), pl.BlockSpec((br, H), row),
                  pl.BlockSpec((br, H), row), pl.BlockSpec((1, H), fixed),
                  pl.BlockSpec((H, H), fixed)],
        out_specs=pl.BlockSpec((br, H), row),
        out_shape=jax.ShapeDtypeStruct((r, H), jnp.float32),
    )(x, a0, a1, c, w)


def _armm(x, m_in, v, g, bvec, w, br):
    r = x.shape[0]
    row = lambda i: (i, 0)
    fixed = lambda i: (0, 0)
    return pl.pallas_call(
        _armm_body,
        grid=(r // br,),
        in_specs=[pl.BlockSpec((br, H), row)]
        + [pl.BlockSpec((1, H), fixed)] * 4
        + [pl.BlockSpec((H, H), fixed)],
        out_specs=pl.BlockSpec((br, H), row),
        out_shape=jax.ShapeDtypeStruct((r, H), jnp.float32),
    )(x, m_in, v, g, bvec, w)


def _bn_relu_body(x_ref, m_ref, v_ref, g_ref, b_ref, o_ref):
    o_ref[...] = jnp.maximum(
        g_ref[...] * (x_ref[...] - m_ref[...])
        / jnp.sqrt(v_ref[...] + 1e-5) + b_ref[...], 0.0)


def _bn_relu(x, m_in, v, g, bvec, br):
    r = x.shape[0]
    row = lambda i: (i, 0)
    fixed = lambda i: (0, 0)
    return pl.pallas_call(
        _bn_relu_body,
        grid=(r // br,),
        in_specs=[pl.BlockSpec((br, H), row)]
        + [pl.BlockSpec((1, H), fixed)] * 4,
        out_specs=pl.BlockSpec((br, H), row),
        out_shape=jax.ShapeDtypeStruct((r, H), jnp.float32),
    )(x, m_in, v, g, bvec)


def _msg_relu_body(a_ref, b_ref, o_ref):
    o_ref[...] = jnp.maximum(a_ref[...] + b_ref[...], 0.0)


def _msg_relu(a, b, br):
    r = a.shape[0]
    row = lambda i: (i, 0)
    return pl.pallas_call(
        _msg_relu_body,
        grid=(r // br,),
        in_specs=[pl.BlockSpec((br, H), row), pl.BlockSpec((br, H), row)],
        out_specs=pl.BlockSpec((br, H), row),
        out_shape=jax.ShapeDtypeStruct((r, H), jnp.float32),
    )(a, b)


def _readout_body(gx_ref, ge_ref, wa_ref, wb_ref, g1_ref, b1_ref, w12_ref,
                  g2_ref, b2_ref, w21_ref, h1g_ref, h1b_ref, w22_ref,
                  h2g_ref, h2b_ref, wf_ref, o_ref):
    def bn_relu(t, g, b):
        m = jnp.mean(t, axis=0, keepdims=True)
        v = jnp.mean((t - m) ** 2, axis=0, keepdims=True)
        return jnp.maximum(g * (t - m) / jnp.sqrt(v + 1e-5) + b, 0.0)

    t = (jnp.dot(gx_ref[...], wa_ref[...], preferred_element_type=jnp.float32)
         + jnp.dot(ge_ref[...], wb_ref[...], preferred_element_type=jnp.float32))
    t = bn_relu(t, g1_ref[...], b1_ref[...])
    t = jnp.dot(t, w12_ref[...], preferred_element_type=jnp.float32)
    t = bn_relu(t, g2_ref[...], b2_ref[...])
    t = jnp.dot(t, w21_ref[...], preferred_element_type=jnp.float32)
    t = bn_relu(t, h1g_ref[...], h1b_ref[...])
    t = jnp.dot(t, w22_ref[...], preferred_element_type=jnp.float32)
    t = bn_relu(t, h2g_ref[...], h2b_ref[...])
    o_ref[...] = jnp.dot(t, wf_ref[...], preferred_element_type=jnp.float32)


def _readout(gx, ge, wa, wb, g1, b1, w12, g2, b2, w21, h1g, h1b, w22, h2g,
             h2b, wf):
    args = (gx, ge, wa, wb, g1.reshape(1, H), b1.reshape(1, H), w12,
            g2.reshape(1, H), b2.reshape(1, H), w21, h1g.reshape(1, H),
            h1b.reshape(1, H), w22, h2g.reshape(1, H), h2b.reshape(1, H), wf)
    return pl.pallas_call(
        _readout_body,
        out_shape=jax.ShapeDtypeStruct((NG, 1), jnp.float32),
    )(*args)


def _sc_mesh():
    return plsc.VectorSubcoreMesh(core_axis_name="c", subcore_axis_name="s")


@functools.cache
def _make_sc_gather(total, t_active, k_chunks, chunk):
    per = k_chunks * chunk

    @functools.partial(
        pl.kernel,
        out_type=jax.ShapeDtypeStruct((total, H), jnp.float32),
        mesh=_sc_mesh(),
        scratch_types=[pltpu.VMEM((chunk,), jnp.int32),
                       pltpu.VMEM((chunk, H), jnp.float32),
                       pltpu.SemaphoreType.DMA],
    )
    def gather_k(table, idx, out, idx_v, rows_v, sem):
        cid = lax.axis_index("c")
        sid = lax.axis_index("s")
        wid = sid * NCORES + cid

        @pl.when(wid < t_active)
        def _():
            def body(i, carry):
                b = wid * per + i * chunk
                pltpu.sync_copy(idx.at[pl.ds(b, chunk)], idx_v)
                pltpu.async_copy(table.at[idx_v], rows_v, sem).wait()
                pltpu.sync_copy(rows_v, out.at[pl.ds(b, chunk)])
                return carry

            lax.fori_loop(0, k_chunks, body, 0)

    return gather_k


@functools.cache
def _make_sc_scatter_add(n_out, total, t_active, k_chunks, chunk):
    per = k_chunks * chunk
    rpt = (n_out // NSUB) & ~7
    rem = n_out - NSUB * rpt

    @functools.partial(
        pl.kernel,
        out_type=jax.ShapeDtypeStruct((NCORES * n_out, H), jnp.float32),
        mesh=_sc_mesh(),
        scratch_types=[pltpu.VMEM((chunk,), jnp.int32),
                       pltpu.VMEM((chunk, H), jnp.float32),
                       pltpu.VMEM_SHARED((n_out, H), jnp.float32)],
    )
    def scatter_k(vals, idx, zeros, out, idx_v, val_v, acc):
        cid = lax.axis_index("c")
        sid = lax.axis_index("s")
        wid = sid * NCORES + cid
        pltpu.sync_copy(zeros.at[pl.ds(sid * rpt, rpt)],
                        acc.at[pl.ds(sid * rpt, rpt)])
        if rem:
            @pl.when(sid == NSUB - 1)
            def _():
                pltpu.sync_copy(zeros.at[pl.ds(NSUB * rpt, rem)],
                                acc.at[pl.ds(NSUB * rpt, rem)])
        plsc.subcore_barrier()

        @pl.when(wid < t_active)
        def _():
            def body(i, carry):
                b = wid * per + i * chunk
                pltpu.sync_copy(idx.at[pl.ds(b, chunk)], idx_v)
                pltpu.sync_copy(vals.at[pl.ds(b, chunk)], val_v)
                pltpu.sync_copy(val_v, acc.at[idx_v], add=True)
                return carry

            lax.fori_loop(0, k_chunks, body, 0)

        plsc.subcore_barrier()
        pltpu.sync_copy(acc.at[pl.ds(sid * rpt, rpt)],
                        out.at[pl.ds(cid * n_out + sid * rpt, rpt)])
        if rem:
            @pl.when(sid == NSUB - 1)
            def _():
                pltpu.sync_copy(
                    acc.at[pl.ds(NSUB * rpt, rem)],
                    out.at[pl.ds(cid * n_out + NSUB * rpt, rem)])

    return scatter_k


def _bn_coef(h, g, b):
    m = jnp.mean(h, axis=0, keepdims=True)
    v = jnp.var(h, axis=0, keepdims=True)
    return m, v, g.reshape(1, H), b.reshape(1, H)


def kernel(x_idx, edge_index, edge_attr_idx, batch, node_emb, edge_emb,
           nW1, nG1, nB1, nW2, nG2, nB2,
           eW1, eG1, eB1, eW2, eG2, eB2, eps_arr,
           L1W1, L1g1, L1b1, L1W2, L1g2, L1b2,
           L2W1, L2g1, L2b1, L2W2, L2g2, L2b2, Wf):
    src1 = edge_index[0].astype(jnp.int32)
    dst1 = edge_index[1].astype(jnp.int32)
    eidx1 = edge_attr_idx.astype(jnp.int32)
    xidx1 = x_idx.astype(jnp.int32)
    batch1 = batch.astype(jnp.int32)
    zN = jnp.zeros((N, H), jnp.float32)
    zG = jnp.zeros((NG, H), jnp.float32)

    _gather_e = _make_sc_gather(E, 25, 50, 128)
    _gather_n = _make_sc_gather(N, 25, 5, 80)
    _scatter_e2n = _make_sc_scatter_add(N, E, 25, 50, 128)
    _scatter_pool = _make_sc_scatter_add(NG, N, 25, 5, 80)

    x = _gather_n(node_emb, xidx1)
    e = _gather_e(edge_emb, eidx1)
    xs = _gather_e(x, src1)

    for l in range(5):
        c = jnp.full((1, H), 1.0, jnp.float32) + eps_arr[l]
        msg = _msg_relu(xs, e, br=2000)
        aggp = _scatter_e2n(msg, dst1, zN)
        h1 = _mm3(x, aggp[:N], aggp[N:], c, nW1[l], br=2000)
        h2 = _armm(h1, *_bn_coef(h1, nG1[l], nB1[l]), nW2[l], br=2000)
        x = _bn_relu(h2, *_bn_coef(h2, nG2[l], nB2[l]), br=2000)
        xs = _gather_e(x, src1)
        xd = _gather_e(x, dst1)
        f1 = _mm3(e, xs, xd, c, eW1[l], br=2000)
        f2 = _armm(f1, *_bn_coef(f1, eG1[l], eB1[l]), eW2[l], br=2000)
        e = _bn_relu(f2, *_bn_coef(f2, eG2[l], eB2[l]), br=2000)

    e2np = _scatter_e2n(e, dst1, zN)
    e2n = e2np[:N] + e2np[N:]
    gxp = _scatter_pool(x, batch1, zG)
    gep = _scatter_pool(e2n, batch1, zG)
    gx = gxp[:NG] + gxp[NG:]
    ge = gep[:NG] + gep[NG:]
    return _readout(gx, ge, L1W1[:H], L1W1[H:], L1g1, L1b1, L1W2, L1g2,
                    L1b2, L2W1, L2g1, L2b1, L2W2, L2g2, L2b2, Wf)
